# Initial kernel scaffold; baseline (speedup 1.0000x reference)
import jax
import jax.numpy as jnp
from jax.experimental import pallas as pl


def kernel(z_e, W, b):
    raise NotImplementedError("write your pallas kernel here")


# speedup vs baseline: 4.4720x; 4.4720x over previous
import jax
import jax.numpy as jnp
from jax.experimental import pallas as pl
from jax.experimental.pallas import tpu as pltpu

_DIM = 256
_ATOMS = 8192
_BN = 256


def _linear_softmax_kernel(x_ref, w_ref, b_ref, o_ref):
    x = x_ref[...]
    w = w_ref[...]
    logits = jax.lax.dot_general(
        x, w, (((1,), (1,)), ((), ())), preferred_element_type=jnp.float32
    )
    logits = logits + b_ref[...]
    m = jnp.max(logits, axis=1, keepdims=True)
    e = jnp.exp(logits - m)
    o_ref[...] = e / jnp.sum(e, axis=1, keepdims=True)


def kernel(z_e, W, b):
    B, C, H, Wd = z_e.shape
    N = B * H * Wd
    x = jnp.transpose(z_e, (0, 2, 3, 1)).reshape(N, C)
    b2 = b.reshape(1, _ATOMS)
    return pl.pallas_call(
        _linear_softmax_kernel,
        grid=(N // _BN,),
        in_specs=[
            pl.BlockSpec((_BN, C), lambda i: (i, 0)),
            pl.BlockSpec((_ATOMS, C), lambda i: (0, 0)),
            pl.BlockSpec((1, _ATOMS), lambda i: (0, 0)),
        ],
        out_specs=pl.BlockSpec((_BN, _ATOMS), lambda i: (i, 0)),
        out_shape=jax.ShapeDtypeStruct((N, _ATOMS), jnp.float32),
        compiler_params=pltpu.CompilerParams(
            dimension_semantics=("parallel",),
        ),
    )(x, W, b2)
